# baseline (speedup 1.0000x reference)
import functools

import jax
import jax.numpy as jnp
from jax import lax
from jax.experimental import pallas as pl
from jax.experimental.pallas import tpu as pltpu
from jax.experimental.pallas import tpu_sc as plsc

N = 10000
F = 128
H = 128
D = 64
G = 64
E = 320000
EPS = 1e-5

NPAD = 10240
ROWS_PER_TILE = NPAD // 16
EPT = E // 32
CHUNKS = 160
EPAD = 16 * CHUNKS * 128


def _mesh():
    return plsc.VectorSubcoreMesh(core_axis_name="c", subcore_axis_name="s")


def _deg_body(dst_hbm, degp_out, idx_v, deg_v, tmp_v, acc_v, slab):
    c = lax.axis_index("c")
    s = lax.axis_index("s")
    wid = s * 2 + c
    zeros16 = jnp.zeros((16,), jnp.float32)
    ones16 = jnp.ones((16,), jnp.float32)

    def zero_deg(i, _):
        deg_v[pl.ds(i * 16, 16)] = zeros16
        return 0
    lax.fori_loop(0, NPAD // 16, zero_deg, 0)

    pltpu.sync_copy(dst_hbm.at[pl.ds(wid * EPT, EPT)], idx_v)

    def hist(i, _):
        idx = idx_v[pl.ds(i * 16, 16)]
        plsc.addupdate_scatter(deg_v, [idx], ones16)
        return 0
    lax.fori_loop(0, EPT // 16, hist, 0)

    pltpu.sync_copy(deg_v, slab.at[s])
    plsc.subcore_barrier()

    def zero_acc(i, _):
        acc_v[pl.ds(i * 16, 16)] = zeros16
        return 0
    lax.fori_loop(0, ROWS_PER_TILE // 16, zero_acc, 0)

    def reduce_row(r, _):
        pltpu.sync_copy(slab.at[r, pl.ds(s * ROWS_PER_TILE, ROWS_PER_TILE)], tmp_v)

        def add(i, _):
            acc_v[pl.ds(i * 16, 16)] = (
                acc_v[pl.ds(i * 16, 16)] + tmp_v[pl.ds(i * 16, 16)])
            return 0
        lax.fori_loop(0, ROWS_PER_TILE // 16, add, 0)
        return 0
    lax.fori_loop(0, 16, reduce_row, 0)

    pltpu.sync_copy(acc_v, degp_out.at[c, pl.ds(s * ROWS_PER_TILE, ROWS_PER_TILE)])


def _deg_call(dst):
    return pl.kernel(
        _deg_body,
        out_type=jax.ShapeDtypeStruct((2, NPAD), jnp.float32),
        mesh=_mesh(),
        compiler_params=pltpu.CompilerParams(needs_layout_passes=False, disable_bounds_checks=True, disable_semaphore_checks=True),
        scratch_types=[
            pltpu.VMEM((EPT,), jnp.int32),
            pltpu.VMEM((NPAD,), jnp.float32),
            pltpu.VMEM((ROWS_PER_TILE,), jnp.float32),
            pltpu.VMEM((ROWS_PER_TILE,), jnp.float32),
            pltpu.VMEM_SHARED((16, NPAD), jnp.float32),
        ],
    )(dst)


_R = 5
_LEAD = 2
_IBLK = 20
_NBLK = CHUNKS // _IBLK
_RINGS = _IBLK // _R


def _agg_body(hs, src2d, dst2d, out, sidx, didx, gbuf, table, acc, gsem, ssem):
    c = lax.axis_index("c")
    s = lax.axis_index("s")
    rows = pl.ds(s * ROWS_PER_TILE, ROWS_PER_TILE)
    tbl_hbm = hs.at[c]
    pltpu.sync_copy(tbl_hbm.at[rows], table.at[rows])
    pltpu.sync_copy(tbl_hbm.at[rows], acc.at[rows])
    plsc.subcore_barrier()

    def gather(j, slot, src):
        pltpu.async_copy(src.at[sidx.at[j]], gbuf.at[slot], gsem.at[slot])

    def wait_gather(j, slot, src):
        pltpu.make_async_copy(src.at[sidx.at[j]], gbuf.at[slot],
                              gsem.at[slot]).wait()

    def scatter(j, slot):
        pltpu.async_copy(gbuf.at[slot], acc.at[didx.at[j]], ssem.at[slot],
                         add=True)

    def wait_scatter(j, slot):
        pltpu.make_async_copy(gbuf.at[slot], acc.at[didx.at[j]],
                              ssem.at[slot]).wait()

    def block(blk, _):
        base_rows = s * CHUNKS + blk * _IBLK
        pltpu.sync_copy(src2d.at[pl.ds(base_rows, _IBLK)], sidx)
        pltpu.sync_copy(dst2d.at[pl.ds(base_rows, _IBLK)], didx)
        for r in range(_LEAD):
            gather(r, r, table)

        def ring(q, _):
            base = q * _R
            for r in range(_R):
                j = base + r
                wait_gather(j, r, table)
                scatter(j, r)

                @pl.when(j >= _R - _LEAD)
                def _():
                    wait_scatter(j - (_R - _LEAD), (r - (_R - _LEAD)) % _R)

                @pl.when(j + _LEAD < _IBLK)
                def _():
                    gather(j + _LEAD, (r + _LEAD) % _R, table)
            return 0
        lax.fori_loop(0, _RINGS, ring, 0)
        for t in range(_R - _LEAD):
            wait_scatter(_IBLK - (_R - _LEAD) + t, (_IBLK - (_R - _LEAD) + t) % _R)
        return 0
    lax.fori_loop(0, _NBLK, block, 0)

    plsc.subcore_barrier()
    pltpu.sync_copy(acc.at[rows], out.at[c, rows])


def _agg_call(hs, src2d, dst2d, w):
    return pl.kernel(
        _agg_body,
        out_type=jax.ShapeDtypeStruct((2, NPAD, w), jnp.float32),
        mesh=_mesh(),
        compiler_params=pltpu.CompilerParams(use_tc_tiling_on_sc=False, disable_bounds_checks=True, disable_semaphore_checks=True),
        scratch_types=[
            pltpu.VMEM((_IBLK, 128), jnp.int32),
            pltpu.VMEM((_IBLK, 128), jnp.int32),
            pltpu.VMEM((_R, 128, w), jnp.float32),
            pltpu.VMEM_SHARED((NPAD, w), jnp.float32),
            pltpu.VMEM_SHARED((NPAD, w), jnp.float32),
            pltpu.SemaphoreType.DMA((_R,)),
            pltpu.SemaphoreType.DMA((_R,)),
        ],
    )(hs, src2d, dst2d)


def _split_pad(h):
    half = h.shape[1] // 2
    hp = jnp.concatenate(
        [h, jnp.zeros((NPAD - N, h.shape[1]), jnp.float32)], axis=0)
    return jnp.stack([hp[:, :half], hp[:, half:]])


def _k1_body(x_ref, w_ref, degp_ref, hs_out, dinv_out):
    deg = degp_ref[0] + degp_ref[1] + 1.0
    dinv = lax.rsqrt(deg)
    dinv_out[...] = dinv
    h = jnp.dot(x_ref[...], w_ref[...], preferred_element_type=jnp.float32)
    hs_out[...] = _split_pad(h * dinv[:N, None])


def _k1_call(x, W1, degp):
    return pl.pallas_call(
        _k1_body,
        out_shape=(
            jax.ShapeDtypeStruct((2, NPAD, H // 2), jnp.float32),
            jax.ShapeDtypeStruct((NPAD,), jnp.float32),
        ),
    )(x, W1, degp)


def _bn_mm_body(aggs_ref, dinv_ref, b_ref, g_ref, be_ref, w_ref, hs_out):
    dinv = dinv_ref[...][:N, None]
    a = jnp.concatenate([aggs_ref[0], aggs_ref[1]], axis=1)[:N]
    a = a * dinv + b_ref[...]
    m = jnp.mean(a, axis=0)
    v = jnp.mean((a - m) ** 2, axis=0)
    y = g_ref[...] * (a - m) * lax.rsqrt(v + EPS) + be_ref[...]
    y = jnp.maximum(y, 0.0)
    h = jnp.dot(y, w_ref[...], preferred_element_type=jnp.float32)
    hs_out[...] = _split_pad(h * dinv)


def _bn_mm_call(aggs, dinv, b, g, be, W):
    w_out = W.shape[1]
    return pl.pallas_call(
        _bn_mm_body,
        out_shape=jax.ShapeDtypeStruct((2, NPAD, w_out // 2), jnp.float32),
    )(aggs, dinv, b, g, be, W)


def _final_body(aggs_ref, dinv_ref, b_ref, g_ref, be_ref, batch_ref, out_ref):
    a = jnp.concatenate([aggs_ref[0], aggs_ref[1]], axis=1)[:N]
    a = a * dinv_ref[...][:N, None] + b_ref[...]
    m = jnp.mean(a, axis=0)
    v = jnp.mean((a - m) ** 2, axis=0)
    y = g_ref[...] * (a - m) * lax.rsqrt(v + EPS) + be_ref[...]
    gids = lax.broadcasted_iota(jnp.int32, (G, N), 0)
    onehot = (gids == batch_ref[...][None, :]).astype(jnp.float32)
    sums = jnp.dot(onehot, y, preferred_element_type=jnp.float32)
    cnt = jnp.sum(onehot, axis=1, keepdims=True)
    mean = sums / jnp.maximum(cnt, 1.0)
    nrm = jnp.sqrt(jnp.sum(mean * mean, axis=1, keepdims=True))
    out_ref[...] = mean / jnp.maximum(nrm, 1e-12)


def _final_call(aggs, dinv, b, g, be, batch):
    return pl.pallas_call(
        _final_body,
        out_shape=jax.ShapeDtypeStruct((G, D), jnp.float32),
    )(aggs, dinv, b, g, be, batch)


def kernel(x, edge_index, batch, W1, b1, W2, b2, W3, b3,
           g1, be1, g2, be2, g3, be3):
    src = edge_index[0].astype(jnp.int32)
    dst = edge_index[1].astype(jnp.int32)
    pad = EPAD - E
    src2d = jnp.concatenate([src, jnp.zeros((pad,), jnp.int32)]).reshape(-1, 128)
    dst2d = jnp.concatenate([dst, jnp.full((pad,), N, jnp.int32)]).reshape(-1, 128)

    degp = _deg_call(dst)
    hs1, dinv = _k1_call(x, W1, degp)
    agg1 = _agg_call(hs1, src2d, dst2d, H // 2)
    hs2 = _bn_mm_call(agg1, dinv, b1, g1, be1, W2)
    agg2 = _agg_call(hs2, src2d, dst2d, H // 2)
    hs3 = _bn_mm_call(agg2, dinv, b2, g2, be2, W3)
    agg3 = _agg_call(hs3, src2d, dst2d, D // 2)
    return _final_call(agg3, dinv, b3, g3, be3, batch)


# speedup vs baseline: 1.0049x; 1.0049x over previous
import functools

import jax
import jax.numpy as jnp
from jax import lax
from jax.experimental import pallas as pl
from jax.experimental.pallas import tpu as pltpu
from jax.experimental.pallas import tpu_sc as plsc

N = 10000
F = 128
H = 128
D = 64
G = 64
E = 320000
EPS = 1e-5

NPAD = 10240
ROWS_PER_TILE = NPAD // 16
EPT = E // 32
CHUNKS = 160
EPAD = 16 * CHUNKS * 128


def _mesh():
    return plsc.VectorSubcoreMesh(core_axis_name="c", subcore_axis_name="s")


def _deg_body(dst_hbm, degp_out, idx_v, deg_v, tmp_v, acc_v, slab):
    c = lax.axis_index("c")
    s = lax.axis_index("s")
    wid = s * 2 + c
    zeros16 = jnp.zeros((16,), jnp.float32)
    ones16 = jnp.ones((16,), jnp.float32)

    def zero_deg(i, _):
        deg_v[pl.ds(i * 16, 16)] = zeros16
        return 0
    lax.fori_loop(0, NPAD // 16, zero_deg, 0)

    pltpu.sync_copy(dst_hbm.at[pl.ds(wid * EPT, EPT)], idx_v)

    def hist(i, _):
        idx = idx_v[pl.ds(i * 16, 16)]
        plsc.addupdate_scatter(deg_v, [idx], ones16)
        return 0
    lax.fori_loop(0, EPT // 16, hist, 0)

    pltpu.sync_copy(deg_v, slab.at[s])
    plsc.subcore_barrier()

    def zero_acc(i, _):
        acc_v[pl.ds(i * 16, 16)] = zeros16
        return 0
    lax.fori_loop(0, ROWS_PER_TILE // 16, zero_acc, 0)

    def reduce_row(r, _):
        pltpu.sync_copy(slab.at[r, pl.ds(s * ROWS_PER_TILE, ROWS_PER_TILE)], tmp_v)

        def add(i, _):
            acc_v[pl.ds(i * 16, 16)] = (
                acc_v[pl.ds(i * 16, 16)] + tmp_v[pl.ds(i * 16, 16)])
            return 0
        lax.fori_loop(0, ROWS_PER_TILE // 16, add, 0)
        return 0
    lax.fori_loop(0, 16, reduce_row, 0)

    pltpu.sync_copy(acc_v, degp_out.at[c, pl.ds(s * ROWS_PER_TILE, ROWS_PER_TILE)])


def _deg_call(dst):
    return pl.kernel(
        _deg_body,
        out_type=jax.ShapeDtypeStruct((2, NPAD), jnp.float32),
        mesh=_mesh(),
        compiler_params=pltpu.CompilerParams(needs_layout_passes=False, disable_bounds_checks=True, disable_semaphore_checks=True),
        scratch_types=[
            pltpu.VMEM((EPT,), jnp.int32),
            pltpu.VMEM((NPAD,), jnp.float32),
            pltpu.VMEM((ROWS_PER_TILE,), jnp.float32),
            pltpu.VMEM((ROWS_PER_TILE,), jnp.float32),
            pltpu.VMEM_SHARED((16, NPAD), jnp.float32),
        ],
    )(dst)


_R = 5
_LEAD = 3
_IBLK = 20
_NBLK = CHUNKS // _IBLK
_RINGS = _IBLK // _R


def _agg_body(hs, src2d, dst2d, out, sidx, didx, gbuf, table, acc, gsem, ssem):
    c = lax.axis_index("c")
    s = lax.axis_index("s")
    rows = pl.ds(s * ROWS_PER_TILE, ROWS_PER_TILE)
    tbl_hbm = hs.at[c]
    pltpu.sync_copy(tbl_hbm.at[rows], table.at[rows])
    pltpu.sync_copy(tbl_hbm.at[rows], acc.at[rows])
    plsc.subcore_barrier()

    def gather(j, slot, src):
        pltpu.async_copy(src.at[sidx.at[j]], gbuf.at[slot], gsem.at[slot])

    def wait_gather(j, slot, src):
        pltpu.make_async_copy(src.at[sidx.at[j]], gbuf.at[slot],
                              gsem.at[slot]).wait()

    def scatter(j, slot):
        pltpu.async_copy(gbuf.at[slot], acc.at[didx.at[j]], ssem.at[slot],
                         add=True)

    def wait_scatter(j, slot):
        pltpu.make_async_copy(gbuf.at[slot], acc.at[didx.at[j]],
                              ssem.at[slot]).wait()

    def block(blk, _):
        base_rows = s * CHUNKS + blk * _IBLK
        pltpu.sync_copy(src2d.at[pl.ds(base_rows, _IBLK)], sidx)
        pltpu.sync_copy(dst2d.at[pl.ds(base_rows, _IBLK)], didx)
        for r in range(_LEAD):
            gather(r, r, table)

        def ring(q, _):
            base = q * _R
            for r in range(_R):
                j = base + r
                wait_gather(j, r, table)
                scatter(j, r)

                @pl.when(j >= _R - _LEAD)
                def _():
                    wait_scatter(j - (_R - _LEAD), (r - (_R - _LEAD)) % _R)

                @pl.when(j + _LEAD < _IBLK)
                def _():
                    gather(j + _LEAD, (r + _LEAD) % _R, table)
            return 0
        lax.fori_loop(0, _RINGS, ring, 0)
        for t in range(_R - _LEAD):
            wait_scatter(_IBLK - (_R - _LEAD) + t, (_IBLK - (_R - _LEAD) + t) % _R)
        return 0
    lax.fori_loop(0, _NBLK, block, 0)

    plsc.subcore_barrier()
    pltpu.sync_copy(acc.at[rows], out.at[c, rows])


def _agg_call(hs, src2d, dst2d, w):
    return pl.kernel(
        _agg_body,
        out_type=jax.ShapeDtypeStruct((2, NPAD, w), jnp.float32),
        mesh=_mesh(),
        compiler_params=pltpu.CompilerParams(use_tc_tiling_on_sc=False, disable_bounds_checks=True, disable_semaphore_checks=True),
        scratch_types=[
            pltpu.VMEM((_IBLK, 128), jnp.int32),
            pltpu.VMEM((_IBLK, 128), jnp.int32),
            pltpu.VMEM((_R, 128, w), jnp.float32),
            pltpu.VMEM_SHARED((NPAD, w), jnp.float32),
            pltpu.VMEM_SHARED((NPAD, w), jnp.float32),
            pltpu.SemaphoreType.DMA((_R,)),
            pltpu.SemaphoreType.DMA((_R,)),
        ],
    )(hs, src2d, dst2d)


def _split_pad(h):
    half = h.shape[1] // 2
    hp = jnp.concatenate(
        [h, jnp.zeros((NPAD - N, h.shape[1]), jnp.float32)], axis=0)
    return jnp.stack([hp[:, :half], hp[:, half:]])


def _k1_body(x_ref, w_ref, degp_ref, hs_out, dinv_out):
    deg = degp_ref[0] + degp_ref[1] + 1.0
    dinv = lax.rsqrt(deg)
    dinv_out[...] = dinv
    h = jnp.dot(x_ref[...], w_ref[...], preferred_element_type=jnp.float32)
    hs_out[...] = _split_pad(h * dinv[:N, None])


def _k1_call(x, W1, degp):
    return pl.pallas_call(
        _k1_body,
        out_shape=(
            jax.ShapeDtypeStruct((2, NPAD, H // 2), jnp.float32),
            jax.ShapeDtypeStruct((NPAD,), jnp.float32),
        ),
    )(x, W1, degp)


def _bn_mm_body(aggs_ref, dinv_ref, b_ref, g_ref, be_ref, w_ref, hs_out):
    dinv = dinv_ref[...][:N, None]
    a = jnp.concatenate([aggs_ref[0], aggs_ref[1]], axis=1)[:N]
    a = a * dinv + b_ref[...]
    m = jnp.mean(a, axis=0)
    v = jnp.mean((a - m) ** 2, axis=0)
    y = g_ref[...] * (a - m) * lax.rsqrt(v + EPS) + be_ref[...]
    y = jnp.maximum(y, 0.0)
    h = jnp.dot(y, w_ref[...], preferred_element_type=jnp.float32)
    hs_out[...] = _split_pad(h * dinv)


def _bn_mm_call(aggs, dinv, b, g, be, W):
    w_out = W.shape[1]
    return pl.pallas_call(
        _bn_mm_body,
        out_shape=jax.ShapeDtypeStruct((2, NPAD, w_out // 2), jnp.float32),
    )(aggs, dinv, b, g, be, W)


def _final_body(aggs_ref, dinv_ref, b_ref, g_ref, be_ref, batch_ref, out_ref):
    a = jnp.concatenate([aggs_ref[0], aggs_ref[1]], axis=1)[:N]
    a = a * dinv_ref[...][:N, None] + b_ref[...]
    m = jnp.mean(a, axis=0)
    v = jnp.mean((a - m) ** 2, axis=0)
    y = g_ref[...] * (a - m) * lax.rsqrt(v + EPS) + be_ref[...]
    gids = lax.broadcasted_iota(jnp.int32, (G, N), 0)
    onehot = (gids == batch_ref[...][None, :]).astype(jnp.float32)
    sums = jnp.dot(onehot, y, preferred_element_type=jnp.float32)
    cnt = jnp.sum(onehot, axis=1, keepdims=True)
    mean = sums / jnp.maximum(cnt, 1.0)
    nrm = jnp.sqrt(jnp.sum(mean * mean, axis=1, keepdims=True))
    out_ref[...] = mean / jnp.maximum(nrm, 1e-12)


def _final_call(aggs, dinv, b, g, be, batch):
    return pl.pallas_call(
        _final_body,
        out_shape=jax.ShapeDtypeStruct((G, D), jnp.float32),
    )(aggs, dinv, b, g, be, batch)


def kernel(x, edge_index, batch, W1, b1, W2, b2, W3, b3,
           g1, be1, g2, be2, g3, be3):
    src = edge_index[0].astype(jnp.int32)
    dst = edge_index[1].astype(jnp.int32)
    pad = EPAD - E
    src2d = jnp.concatenate([src, jnp.zeros((pad,), jnp.int32)]).reshape(-1, 128)
    dst2d = jnp.concatenate([dst, jnp.full((pad,), N, jnp.int32)]).reshape(-1, 128)

    degp = _deg_call(dst)
    hs1, dinv = _k1_call(x, W1, degp)
    agg1 = _agg_call(hs1, src2d, dst2d, H // 2)
    hs2 = _bn_mm_call(agg1, dinv, b1, g1, be1, W2)
    agg2 = _agg_call(hs2, src2d, dst2d, H // 2)
    hs3 = _bn_mm_call(agg2, dinv, b2, g2, be2, W3)
    agg3 = _agg_call(hs3, src2d, dst2d, D // 2)
    return _final_call(agg3, dinv, b3, g3, be3, batch)


# speedup vs baseline: 1.0072x; 1.0023x over previous
import functools

import jax
import jax.numpy as jnp
from jax import lax
from jax.experimental import pallas as pl
from jax.experimental.pallas import tpu as pltpu
from jax.experimental.pallas import tpu_sc as plsc

N = 10000
F = 128
H = 128
D = 64
G = 64
E = 320000
EPS = 1e-5

NPAD = 10240
ROWS_PER_TILE = NPAD // 16
EPT = E // 32
CW = 64
CHUNKS = 320
EPAD = 16 * CHUNKS * CW


def _mesh():
    return plsc.VectorSubcoreMesh(core_axis_name="c", subcore_axis_name="s")


def _deg_body(dst_hbm, degp_out, idx_v, deg_v, tmp_v, acc_v, slab):
    c = lax.axis_index("c")
    s = lax.axis_index("s")
    wid = s * 2 + c
    zeros16 = jnp.zeros((16,), jnp.float32)
    ones16 = jnp.ones((16,), jnp.float32)

    def zero_deg(i, _):
        deg_v[pl.ds(i * 16, 16)] = zeros16
        return 0
    lax.fori_loop(0, NPAD // 16, zero_deg, 0)

    pltpu.sync_copy(dst_hbm.at[pl.ds(wid * EPT, EPT)], idx_v)

    def hist(i, _):
        idx = idx_v[pl.ds(i * 16, 16)]
        plsc.addupdate_scatter(deg_v, [idx], ones16)
        return 0
    lax.fori_loop(0, EPT // 16, hist, 0)

    pltpu.sync_copy(deg_v, slab.at[s])
    plsc.subcore_barrier()

    def zero_acc(i, _):
        acc_v[pl.ds(i * 16, 16)] = zeros16
        return 0
    lax.fori_loop(0, ROWS_PER_TILE // 16, zero_acc, 0)

    def reduce_row(r, _):
        pltpu.sync_copy(slab.at[r, pl.ds(s * ROWS_PER_TILE, ROWS_PER_TILE)], tmp_v)

        def add(i, _):
            acc_v[pl.ds(i * 16, 16)] = (
                acc_v[pl.ds(i * 16, 16)] + tmp_v[pl.ds(i * 16, 16)])
            return 0
        lax.fori_loop(0, ROWS_PER_TILE // 16, add, 0)
        return 0
    lax.fori_loop(0, 16, reduce_row, 0)

    pltpu.sync_copy(acc_v, degp_out.at[c, pl.ds(s * ROWS_PER_TILE, ROWS_PER_TILE)])


def _deg_call(dst):
    return pl.kernel(
        _deg_body,
        out_type=jax.ShapeDtypeStruct((2, NPAD), jnp.float32),
        mesh=_mesh(),
        compiler_params=pltpu.CompilerParams(needs_layout_passes=False, disable_bounds_checks=True, disable_semaphore_checks=True),
        scratch_types=[
            pltpu.VMEM((EPT,), jnp.int32),
            pltpu.VMEM((NPAD,), jnp.float32),
            pltpu.VMEM((ROWS_PER_TILE,), jnp.float32),
            pltpu.VMEM((ROWS_PER_TILE,), jnp.float32),
            pltpu.VMEM_SHARED((16, NPAD), jnp.float32),
        ],
    )(dst)


_R = 10
_LEAD = 6
_IBLK = 40
_NBLK = CHUNKS // _IBLK
_RINGS = _IBLK // _R


def _agg_body(hs, src2d, dst2d, out, sidx, didx, gbuf, table, acc, gsem, ssem):
    c = lax.axis_index("c")
    s = lax.axis_index("s")
    rows = pl.ds(s * ROWS_PER_TILE, ROWS_PER_TILE)
    tbl_hbm = hs.at[c]
    pltpu.sync_copy(tbl_hbm.at[rows], table.at[rows])
    pltpu.sync_copy(tbl_hbm.at[rows], acc.at[rows])
    plsc.subcore_barrier()

    def gather(j, slot, src):
        pltpu.async_copy(src.at[sidx.at[j]], gbuf.at[slot], gsem.at[slot])

    def wait_gather(j, slot, src):
        pltpu.make_async_copy(src.at[sidx.at[j]], gbuf.at[slot],
                              gsem.at[slot]).wait()

    def scatter(j, slot):
        pltpu.async_copy(gbuf.at[slot], acc.at[didx.at[j]], ssem.at[slot],
                         add=True)

    def wait_scatter(j, slot):
        pltpu.make_async_copy(gbuf.at[slot], acc.at[didx.at[j]],
                              ssem.at[slot]).wait()

    def block(blk, _):
        base_rows = s * CHUNKS + blk * _IBLK
        pltpu.sync_copy(src2d.at[pl.ds(base_rows, _IBLK)], sidx)
        pltpu.sync_copy(dst2d.at[pl.ds(base_rows, _IBLK)], didx)
        for r in range(_LEAD):
            gather(r, r, table)

        def ring(q, _):
            base = q * _R
            for r in range(_R):
                j = base + r
                wait_gather(j, r, table)
                scatter(j, r)

                @pl.when(j >= _R - _LEAD)
                def _():
                    wait_scatter(j - (_R - _LEAD), (r - (_R - _LEAD)) % _R)

                @pl.when(j + _LEAD < _IBLK)
                def _():
                    gather(j + _LEAD, (r + _LEAD) % _R, table)
            return 0
        lax.fori_loop(0, _RINGS, ring, 0)
        for t in range(_R - _LEAD):
            wait_scatter(_IBLK - (_R - _LEAD) + t, (_IBLK - (_R - _LEAD) + t) % _R)
        return 0
    lax.fori_loop(0, _NBLK, block, 0)

    plsc.subcore_barrier()
    pltpu.sync_copy(acc.at[rows], out.at[c, rows])


def _agg_call(hs, src2d, dst2d, w):
    return pl.kernel(
        _agg_body,
        out_type=jax.ShapeDtypeStruct((2, NPAD, w), jnp.float32),
        mesh=_mesh(),
        compiler_params=pltpu.CompilerParams(use_tc_tiling_on_sc=False, disable_bounds_checks=True, disable_semaphore_checks=True),
        scratch_types=[
            pltpu.VMEM((_IBLK, CW), jnp.int32),
            pltpu.VMEM((_IBLK, CW), jnp.int32),
            pltpu.VMEM((_R, CW, w), jnp.float32),
            pltpu.VMEM_SHARED((NPAD, w), jnp.float32),
            pltpu.VMEM_SHARED((NPAD, w), jnp.float32),
            pltpu.SemaphoreType.DMA((_R,)),
            pltpu.SemaphoreType.DMA((_R,)),
        ],
    )(hs, src2d, dst2d)


def _split_pad(h):
    half = h.shape[1] // 2
    hp = jnp.concatenate(
        [h, jnp.zeros((NPAD - N, h.shape[1]), jnp.float32)], axis=0)
    return jnp.stack([hp[:, :half], hp[:, half:]])


def _k1_body(x_ref, w_ref, degp_ref, hs_out, dinv_out):
    deg = degp_ref[0] + degp_ref[1] + 1.0
    dinv = lax.rsqrt(deg)
    dinv_out[...] = dinv
    h = jnp.dot(x_ref[...], w_ref[...], preferred_element_type=jnp.float32)
    hs_out[...] = _split_pad(h * dinv[:N, None])


def _k1_call(x, W1, degp):
    return pl.pallas_call(
        _k1_body,
        out_shape=(
            jax.ShapeDtypeStruct((2, NPAD, H // 2), jnp.float32),
            jax.ShapeDtypeStruct((NPAD,), jnp.float32),
        ),
    )(x, W1, degp)


def _bn_mm_body(aggs_ref, dinv_ref, b_ref, g_ref, be_ref, w_ref, hs_out):
    dinv = dinv_ref[...][:N, None]
    a = jnp.concatenate([aggs_ref[0], aggs_ref[1]], axis=1)[:N]
    a = a * dinv + b_ref[...]
    m = jnp.mean(a, axis=0)
    v = jnp.mean((a - m) ** 2, axis=0)
    y = g_ref[...] * (a - m) * lax.rsqrt(v + EPS) + be_ref[...]
    y = jnp.maximum(y, 0.0)
    h = jnp.dot(y, w_ref[...], preferred_element_type=jnp.float32)
    hs_out[...] = _split_pad(h * dinv)


def _bn_mm_call(aggs, dinv, b, g, be, W):
    w_out = W.shape[1]
    return pl.pallas_call(
        _bn_mm_body,
        out_shape=jax.ShapeDtypeStruct((2, NPAD, w_out // 2), jnp.float32),
    )(aggs, dinv, b, g, be, W)


def _final_body(aggs_ref, dinv_ref, b_ref, g_ref, be_ref, batch_ref, out_ref):
    a = jnp.concatenate([aggs_ref[0], aggs_ref[1]], axis=1)[:N]
    a = a * dinv_ref[...][:N, None] + b_ref[...]
    m = jnp.mean(a, axis=0)
    v = jnp.mean((a - m) ** 2, axis=0)
    y = g_ref[...] * (a - m) * lax.rsqrt(v + EPS) + be_ref[...]
    gids = lax.broadcasted_iota(jnp.int32, (G, N), 0)
    onehot = (gids == batch_ref[...][None, :]).astype(jnp.float32)
    sums = jnp.dot(onehot, y, preferred_element_type=jnp.float32)
    cnt = jnp.sum(onehot, axis=1, keepdims=True)
    mean = sums / jnp.maximum(cnt, 1.0)
    nrm = jnp.sqrt(jnp.sum(mean * mean, axis=1, keepdims=True))
    out_ref[...] = mean / jnp.maximum(nrm, 1e-12)


def _final_call(aggs, dinv, b, g, be, batch):
    return pl.pallas_call(
        _final_body,
        out_shape=jax.ShapeDtypeStruct((G, D), jnp.float32),
    )(aggs, dinv, b, g, be, batch)


def kernel(x, edge_index, batch, W1, b1, W2, b2, W3, b3,
           g1, be1, g2, be2, g3, be3):
    src = edge_index[0].astype(jnp.int32)
    dst = edge_index[1].astype(jnp.int32)
    pad = EPAD - E
    src2d = jnp.concatenate([src, jnp.zeros((pad,), jnp.int32)]).reshape(-1, CW)
    dst2d = jnp.concatenate([dst, jnp.full((pad,), N, jnp.int32)]).reshape(-1, CW)

    degp = _deg_call(dst)
    hs1, dinv = _k1_call(x, W1, degp)
    agg1 = _agg_call(hs1, src2d, dst2d, H // 2)
    hs2 = _bn_mm_call(agg1, dinv, b1, g1, be1, W2)
    agg2 = _agg_call(hs2, src2d, dst2d, H // 2)
    hs3 = _bn_mm_call(agg2, dinv, b2, g2, be2, W3)
    agg3 = _agg_call(hs3, src2d, dst2d, D // 2)
    return _final_call(agg3, dinv, b3, g3, be3, batch)
